# Initial kernel scaffold; baseline (speedup 1.0000x reference)
import jax
import jax.numpy as jnp
from jax.experimental import pallas as pl


def kernel(x, path_list, path_index, path_edge_type, path_positions, path_weight, gamma, beta, W_proj, b_proj, edge_table, pos_table, W1, b1, W2, b2):
    raise NotImplementedError("write your pallas kernel here")


# speedup vs baseline: 60.2289x; 60.2289x over previous
import functools

import jax
import jax.numpy as jnp
from jax import lax
from jax.experimental import pallas as pl
from jax.experimental.pallas import tpu as pltpu
from jax.experimental.pallas import tpu_sc as plsc

B, N, H = 4, 10000, 256
R = 4
EMB = H // R
MAXLEN = 8
NE = 16
T = 32768
P = 4096
KM = (NE + 1) * MAXLEN

NC, NS, L = 2, 16, 16
TSUB = T // NS
CH = TSUB // 128
NV = TSUB // L
NPAD = 10240
ZR = NPAD // NS

BN1 = 1000
BN3 = 1000


def _dense1_body(x_ref, wg_ref, bg_ref, vg4_ref, c16_ref, xp_ref, y2_ref):
    zs = []
    for b in range(B):
        xb = x_ref[b]
        mu = jnp.mean(xb, axis=1, keepdims=True)
        xc = xb - mu
        var = jnp.mean(xc * xc, axis=1, keepdims=True)
        z = xc * lax.rsqrt(var + 1e-5)
        zs.append(z)
        xp_ref[b] = (
            jnp.dot(z, wg_ref[...], preferred_element_type=jnp.float32)
            + bg_ref[...]
        )
    z4 = jnp.concatenate(zs, axis=1)
    y16 = (
        jnp.dot(z4, vg4_ref[...], preferred_element_type=jnp.float32)
        + c16_ref[...]
    )
    y2_ref[0] = y16[:, 0:8]
    y2_ref[1] = y16[:, 8:16]


def _dense1(x, wg, bg, vg4, c16):
    return pl.pallas_call(
        _dense1_body,
        grid=(N // BN1,),
        in_specs=[
            pl.BlockSpec((B, BN1, H), lambda i: (0, i, 0)),
            pl.BlockSpec((H, EMB), lambda i: (0, 0)),
            pl.BlockSpec((1, EMB), lambda i: (0, 0)),
            pl.BlockSpec((B * H, 16), lambda i: (0, 0)),
            pl.BlockSpec((1, 16), lambda i: (0, 0)),
        ],
        out_specs=[
            pl.BlockSpec((B, BN1, EMB), lambda i: (0, i, 0)),
            pl.BlockSpec((2, BN1, 8), lambda i: (0, i, 0)),
        ],
        out_shape=[
            jax.ShapeDtypeStruct((B, N, EMB), jnp.float32),
            jax.ShapeDtypeStruct((2, N, 8), jnp.float32),
        ],
    )(x, wg, bg, vg4, c16)


_mesh = plsc.VectorSubcoreMesh(core_axis_name="c", subcore_axis_name="s")


@functools.partial(
    pl.kernel,
    out_type=(
        jax.ShapeDtypeStruct((NC, NPAD, 8), jnp.float32),
        jax.ShapeDtypeStruct((NC, KM, 8), jnp.float32),
    ),
    mesh=_mesh,
    compiler_params=pltpu.CompilerParams(
        use_tc_tiling_on_sc=False, needs_layout_passes=False),
    scratch_types=dict(
        plv=pltpu.VMEM((CH, 128), jnp.int32),
        piv=pltpu.VMEM((CH, 128), jnp.int32),
        kmv=pltpu.VMEM((CH, 128), jnp.int32),
        t0=pltpu.VMEM((TSUB, 8), jnp.float32),
        t1=pltpu.VMEM((TSUB, 8), jnp.float32),
        t2=pltpu.VMEM((TSUB, 8), jnp.float32),
        t3=pltpu.VMEM((TSUB, 8), jnp.float32),
        w2v=pltpu.VMEM((16, 16), jnp.float32),
        b2v=pltpu.VMEM((4, 16), jnp.float32),
        seg=pltpu.VMEM_SHARED((P, 8), jnp.float32),
        aacc=pltpu.VMEM_SHARED((NPAD, 8), jnp.float32),
        dacc=pltpu.VMEM_SHARED((KM, 8), jnp.float32),
    ),
)
def _sc_sparse(y2, pw2, ewp8, w2rep, b2rep, pl3, pi3, km3, zeros, a_out, d_out,
               plv, piv, kmv, t0, t1, t2, t3, w2v, b2v,
               seg, aacc, dacc):
    cid = lax.axis_index("c")
    sid = lax.axis_index("s")
    iota = lax.broadcasted_iota(jnp.int32, (L,), 0)

    pltpu.sync_copy(w2rep, w2v)
    pltpu.sync_copy(b2rep, b2v)
    pltpu.sync_copy(pl3.at[pl.ds(sid * CH, CH)], plv)
    pltpu.sync_copy(pi3.at[pl.ds(sid * CH, CH)], piv)
    pltpu.sync_copy(km3.at[pl.ds(sid * CH, CH)], kmv)

    pltpu.sync_copy(zeros, aacc.at[pl.ds(sid * ZR, ZR)])
    pltpu.sync_copy(zeros.at[pl.ds(0, P // NS)], seg.at[pl.ds(sid * (P // NS), P // NS)])

    @pl.when(sid == 0)
    def _():
        pltpu.sync_copy(zeros.at[pl.ds(0, KM)], dacc)

    plsc.subcore_barrier()

    def _gather(j, _):
        dst = pl.ds(j * 128, 128)
        pltpu.sync_copy(y2.at[cid].at[plv.at[j]], t0.at[dst])
        pltpu.sync_copy(ewp8.at[kmv.at[j]], t1.at[dst])
        pltpu.sync_copy(pw2.at[cid].at[piv.at[j]], t3.at[dst])
        return 0

    lax.fori_loop(0, CH, _gather, 0)

    for jc in range(8):
        cj = jnp.full((L,), jc, jnp.int32)

        def _pa(v, _):
            ridx = v * L + iota
            pre = plsc.load_gather(t0, [ridx, cj]) + plsc.load_gather(t1, [ridx, cj])
            e2 = jnp.exp(pre * 2.0)
            th = 1.0 - 2.0 / (e2 + 1.0)
            plsc.store_scatter(t0, [ridx, cj], th)
            return 0

        lax.fori_loop(0, NV, _pa, 0)

    for jc in range(8):
        bl4 = (jc // 4) * 4
        r_ = jc % 4
        cj = jnp.full((L,), jc, jnp.int32)
        cps = [jnp.full((L,), bl4 + rp, jnp.int32) for rp in range(4)]

        def _pb(v, _):
            ridx = v * L + iota
            s = b2v[r_]
            for rp in range(4):
                s = s + w2v[4 * r_ + rp] * plsc.load_gather(t0, [ridx, cps[rp]])
            plsc.store_scatter(t1, [ridx, cj], jnp.exp(s))
            return 0

        lax.fori_loop(0, NV, _pb, 0)

    def _segadd(j, _):
        pltpu.sync_copy(t1.at[pl.ds(j * 128, 128)], seg.at[piv.at[j]], add=True)
        return 0

    lax.fori_loop(0, CH, _segadd, 0)
    plsc.subcore_barrier()

    def _seggather(j, _):
        pltpu.sync_copy(seg.at[piv.at[j]], t2.at[pl.ds(j * 128, 128)])
        return 0

    lax.fori_loop(0, CH, _seggather, 0)

    for jc in range(8):
        cj = jnp.full((L,), jc, jnp.int32)

        def _pc(v, _):
            ridx = v * L + iota
            cval = (
                plsc.load_gather(t1, [ridx, cj])
                / plsc.load_gather(t2, [ridx, cj])
                * plsc.load_gather(t3, [ridx, cj])
                * (1.0 / P)
            )
            plsc.store_scatter(t1, [ridx, cj], cval)
            return 0

        lax.fori_loop(0, NV, _pc, 0)

    def _accadd(j, _):
        src = t1.at[pl.ds(j * 128, 128)]
        pltpu.sync_copy(src, aacc.at[plv.at[j]], add=True)
        pltpu.sync_copy(src, dacc.at[kmv.at[j]], add=True)
        return 0

    lax.fori_loop(0, CH, _accadd, 0)
    plsc.subcore_barrier()

    pltpu.sync_copy(aacc.at[pl.ds(sid * ZR, ZR)],
                    a_out.at[cid].at[pl.ds(sid * ZR, ZR)])

    @pl.when(sid == 0)
    def _():
        pltpu.sync_copy(dacc, d_out.at[cid])


def _dense3_body(a_ref, xp_ref, d_ref, ew_ref, o_ref):
    i = pl.program_id(1)
    a = a_ref[0]
    lane = lax.broadcasted_iota(jnp.int32, (BN3, 8), 1)
    acc = jnp.zeros((8, EMB), jnp.float32)
    for bl in range(2):
        m = (lane // 4 == bl).astype(jnp.float32)
        acc = acc + lax.dot_general(
            a * m, xp_ref[bl],
            (((0,), (0,)), ((), ())),
            preferred_element_type=jnp.float32,
        )

    @pl.when(i == 0)
    def _():
        o_ref[0] = lax.dot_general(
            d_ref[0], ew_ref[...],
            (((0,), (0,)), ((), ())),
            preferred_element_type=jnp.float32,
        )

    o_ref[0] += acc


def _dense3(a2, xp, d2, ewfull):
    return pl.pallas_call(
        _dense3_body,
        grid=(2, N // BN3),
        in_specs=[
            pl.BlockSpec((1, BN3, 8), lambda c, i: (c, i, 0)),
            pl.BlockSpec((2, BN3, EMB), lambda c, i: (c, i, 0)),
            pl.BlockSpec((1, KM, 8), lambda c, i: (c, 0, 0)),
            pl.BlockSpec((KM, EMB), lambda c, i: (0, 0)),
        ],
        out_specs=pl.BlockSpec((1, 8, EMB), lambda c, i: (c, 0, 0)),
        out_shape=jax.ShapeDtypeStruct((2, 8, EMB), jnp.float32),
    )(a2, xp, d2, ewfull)


def kernel(x, path_list, path_index, path_edge_type, path_positions,
           path_weight, gamma, beta, W_proj, b_proj, edge_table,
           pos_table, W1, b1, W2, b2):
    f32 = jnp.float32
    wg = (gamma[:, None] * W_proj.T).astype(f32)
    bg = (beta @ W_proj.T + b_proj)[None, :].astype(f32)
    w1t = W1.T
    vg = wg @ w1t
    vg4 = jnp.zeros((B * H, 16), f32)
    for b in range(B):
        vg4 = vg4.at[b * H:(b + 1) * H, b * R:(b + 1) * R].set(vg)
    c16 = jnp.tile(bg @ w1t, (1, B))

    xp, y2 = _dense1(x, wg, bg, vg4, c16)

    ewp = (edge_table @ w1t + b1)[:, None, :] + (pos_table @ w1t)[None, :, :]
    ewp8 = jnp.tile(ewp.reshape(KM, R), (1, 2))
    ewfull = (edge_table[:, None, :] + pos_table[None, :, :]).reshape(KM, EMB)
    pw2 = jnp.repeat(path_weight[:, :, 0].T, R, axis=1).reshape(P, 2, 8)
    pw2 = jnp.transpose(pw2, (1, 0, 2))
    w2rep = jnp.repeat(W2.reshape(16)[:, None], 16, axis=1)
    b2rep = jnp.repeat(b2[:, None], 16, axis=1)

    km = (path_edge_type * MAXLEN + path_positions).astype(jnp.int32)
    pl3 = path_list.astype(jnp.int32).reshape(T // 128, 128)
    pi3 = path_index.astype(jnp.int32).reshape(T // 128, 128)
    km3 = km.reshape(T // 128, 128)

    zeros = jnp.zeros((ZR, 8), f32)
    a2, d2 = _sc_sparse(y2, pw2, ewp8, w2rep, b2rep, pl3, pi3, km3, zeros)
    out8 = _dense3(a2, xp, d2, ewfull)
    return out8.reshape(B, H)


# speedup vs baseline: 78.5842x; 1.3048x over previous
import functools

import jax
import jax.numpy as jnp
from jax import lax
from jax.experimental import pallas as pl
from jax.experimental.pallas import tpu as pltpu
from jax.experimental.pallas import tpu_sc as plsc

B, N, H = 4, 10000, 256
R = 4
EMB = H // R
MAXLEN = 8
NE = 16
T = 32768
P = 4096
KM = (NE + 1) * MAXLEN

NC, NS, L = 2, 16, 16
TSUB = T // NS
CH = TSUB // 128
NV = TSUB // L
NPAD = 10240
ZR = NPAD // NS

BN1 = 1000
BN3 = 1000


def _dense1_body(x_ref, wg_ref, bg_ref, vg4_ref, c16_ref, xp_ref, y2_ref):
    zs = []
    for b in range(B):
        xb = x_ref[b]
        mu = jnp.mean(xb, axis=1, keepdims=True)
        xc = xb - mu
        var = jnp.mean(xc * xc, axis=1, keepdims=True)
        z = xc * lax.rsqrt(var + 1e-5)
        zs.append(z)
        xp_ref[b] = (
            jnp.dot(z, wg_ref[...], preferred_element_type=jnp.float32)
            + bg_ref[...]
        )
    z4 = jnp.concatenate(zs, axis=1)
    y16 = (
        jnp.dot(z4, vg4_ref[...], preferred_element_type=jnp.float32)
        + c16_ref[...]
    )
    y2_ref[0] = y16[:, 0:8]
    y2_ref[1] = y16[:, 8:16]


def _dense1(x, wg, bg, vg4, c16):
    return pl.pallas_call(
        _dense1_body,
        grid=(N // BN1,),
        in_specs=[
            pl.BlockSpec((B, BN1, H), lambda i: (0, i, 0)),
            pl.BlockSpec((H, EMB), lambda i: (0, 0)),
            pl.BlockSpec((1, EMB), lambda i: (0, 0)),
            pl.BlockSpec((B * H, 16), lambda i: (0, 0)),
            pl.BlockSpec((1, 16), lambda i: (0, 0)),
        ],
        out_specs=[
            pl.BlockSpec((B, BN1, EMB), lambda i: (0, i, 0)),
            pl.BlockSpec((2, BN1, 8), lambda i: (0, i, 0)),
        ],
        out_shape=[
            jax.ShapeDtypeStruct((B, N, EMB), jnp.float32),
            jax.ShapeDtypeStruct((2, N, 8), jnp.float32),
        ],
    )(x, wg, bg, vg4, c16)


_mesh = plsc.VectorSubcoreMesh(core_axis_name="c", subcore_axis_name="s")


@functools.partial(
    pl.kernel,
    out_type=(
        jax.ShapeDtypeStruct((NC, NPAD, 8), jnp.float32),
        jax.ShapeDtypeStruct((NC, KM, 8), jnp.float32),
    ),
    mesh=_mesh,
    compiler_params=pltpu.CompilerParams(
        use_tc_tiling_on_sc=False, needs_layout_passes=False),
    scratch_types=dict(
        plv=pltpu.VMEM((TSUB,), jnp.int32),
        piv=pltpu.VMEM((TSUB,), jnp.int32),
        kmv=pltpu.VMEM((TSUB,), jnp.int32),
        t0=pltpu.VMEM((TSUB, 8), jnp.float32),
        t1=pltpu.VMEM((TSUB, 8), jnp.float32),
        t2=pltpu.VMEM((TSUB, 8), jnp.float32),
        t3=pltpu.VMEM((TSUB, 8), jnp.float32),
        w2v=pltpu.VMEM((16, 16), jnp.float32),
        b2v=pltpu.VMEM((4, 16), jnp.float32),
        seg=pltpu.VMEM_SHARED((P, 8), jnp.float32),
        aacc=pltpu.VMEM_SHARED((NPAD, 8), jnp.float32),
        dacc=pltpu.VMEM_SHARED((KM, 8), jnp.float32),
        sem0=pltpu.SemaphoreType.DMA,
        sem1=pltpu.SemaphoreType.DMA,
        sem2=pltpu.SemaphoreType.DMA,
    ),
)
def _sc_sparse(y2, pw2, ewp8, w2rep, b2rep, pl1, pi1, km1, zeros, a_out, d_out,
               plv, piv, kmv, t0, t1, t2, t3, w2v, b2v,
               seg, aacc, dacc, sem0, sem1, sem2):
    cid = lax.axis_index("c")
    sid = lax.axis_index("s")
    iota = lax.broadcasted_iota(jnp.int32, (L,), 0)
    base = sid * TSUB

    c_pl = pltpu.async_copy(pl1.at[pl.ds(base, TSUB)], plv, sem0)
    c_pi = pltpu.async_copy(pi1.at[pl.ds(base, TSUB)], piv, sem1)
    c_km = pltpu.async_copy(km1.at[pl.ds(base, TSUB)], kmv, sem2)
    pltpu.sync_copy(w2rep, w2v)
    pltpu.sync_copy(b2rep, b2v)
    pltpu.sync_copy(zeros, aacc.at[pl.ds(sid * ZR, ZR)])
    pltpu.sync_copy(zeros.at[pl.ds(0, P // NS)], seg.at[pl.ds(sid * (P // NS), P // NS)])

    @pl.when(sid == 0)
    def _():
        pltpu.sync_copy(zeros.at[pl.ds(0, KM)], dacc)

    c_pl.wait()
    c_km.wait()
    c_y = pltpu.async_copy(y2.at[cid].at[plv], t0, sem0)
    c_e = pltpu.async_copy(ewp8.at[kmv], t1, sem2)
    c_pi.wait()
    c_w = pltpu.async_copy(pw2.at[cid].at[piv], t3, sem1)
    plsc.subcore_barrier()
    c_y.wait()
    c_e.wait()

    for bl4 in (0, 4):
        cqs = [jnp.full((L,), bl4 + q, jnp.int32) for q in range(4)]

        def _pab(v, _):
            ridx = v * L + iota
            ths = []
            for q in range(4):
                pre = (plsc.load_gather(t0, [ridx, cqs[q]])
                       + plsc.load_gather(t1, [ridx, cqs[q]]))
                e2 = jnp.exp(pre * 2.0)
                ths.append(1.0 - 2.0 / (e2 + 1.0))
            exs = []
            for r_ in range(4):
                s = b2v[r_]
                for q in range(4):
                    s = s + w2v[4 * r_ + q] * ths[q]
                exs.append(jnp.exp(s))
            for r_ in range(4):
                plsc.store_scatter(t1, [ridx, cqs[r_]], exs[r_])
            return 0

        lax.fori_loop(0, NV, _pab, 0)

    pltpu.sync_copy(t1, seg.at[piv], add=True)
    plsc.subcore_barrier()
    c_s = pltpu.async_copy(seg.at[piv], t2, sem0)
    c_w.wait()
    c_s.wait()

    for jc in range(8):
        cj = jnp.full((L,), jc, jnp.int32)

        def _pc(v, _):
            ridx = v * L + iota
            cval = (
                plsc.load_gather(t1, [ridx, cj])
                / plsc.load_gather(t2, [ridx, cj])
                * plsc.load_gather(t3, [ridx, cj])
                * (1.0 / P)
            )
            plsc.store_scatter(t1, [ridx, cj], cval)
            return 0

        lax.fori_loop(0, NV, _pc, 0)

    pltpu.sync_copy(t1, aacc.at[plv], add=True)
    pltpu.sync_copy(t1, dacc.at[kmv], add=True)
    plsc.subcore_barrier()

    pltpu.sync_copy(aacc.at[pl.ds(sid * ZR, ZR)],
                    a_out.at[cid].at[pl.ds(sid * ZR, ZR)])

    @pl.when(sid == 0)
    def _():
        pltpu.sync_copy(dacc, d_out.at[cid])


def _dense3_body(a_ref, xp_ref, d_ref, ew_ref, o_ref):
    i = pl.program_id(1)
    a = a_ref[0]
    lane = lax.broadcasted_iota(jnp.int32, (BN3, 8), 1)
    acc = jnp.zeros((8, EMB), jnp.float32)
    for bl in range(2):
        m = (lane // 4 == bl).astype(jnp.float32)
        acc = acc + lax.dot_general(
            a * m, xp_ref[bl],
            (((0,), (0,)), ((), ())),
            preferred_element_type=jnp.float32,
        )

    @pl.when(i == 0)
    def _():
        o_ref[0] = lax.dot_general(
            d_ref[0], ew_ref[...],
            (((0,), (0,)), ((), ())),
            preferred_element_type=jnp.float32,
        )

    o_ref[0] += acc


def _dense3(a2, xp, d2, ewfull):
    return pl.pallas_call(
        _dense3_body,
        grid=(2, N // BN3),
        in_specs=[
            pl.BlockSpec((1, BN3, 8), lambda c, i: (c, i, 0)),
            pl.BlockSpec((2, BN3, EMB), lambda c, i: (c, i, 0)),
            pl.BlockSpec((1, KM, 8), lambda c, i: (c, 0, 0)),
            pl.BlockSpec((KM, EMB), lambda c, i: (0, 0)),
        ],
        out_specs=pl.BlockSpec((1, 8, EMB), lambda c, i: (c, 0, 0)),
        out_shape=jax.ShapeDtypeStruct((2, 8, EMB), jnp.float32),
    )(a2, xp, d2, ewfull)


def kernel(x, path_list, path_index, path_edge_type, path_positions,
           path_weight, gamma, beta, W_proj, b_proj, edge_table,
           pos_table, W1, b1, W2, b2):
    f32 = jnp.float32
    wg = (gamma[:, None] * W_proj.T).astype(f32)
    bg = (beta @ W_proj.T + b_proj)[None, :].astype(f32)
    w1t = W1.T
    vg = wg @ w1t
    vg4 = jnp.zeros((B * H, 16), f32)
    for b in range(B):
        vg4 = vg4.at[b * H:(b + 1) * H, b * R:(b + 1) * R].set(vg)
    c16 = jnp.tile(bg @ w1t, (1, B))

    xp, y2 = _dense1(x, wg, bg, vg4, c16)

    ewp = (edge_table @ w1t + b1)[:, None, :] + (pos_table @ w1t)[None, :, :]
    ewp8 = jnp.tile(ewp.reshape(KM, R), (1, 2))
    ewfull = (edge_table[:, None, :] + pos_table[None, :, :]).reshape(KM, EMB)
    pw2 = jnp.repeat(path_weight[:, :, 0].T, R, axis=1).reshape(P, 2, 8)
    pw2 = jnp.transpose(pw2, (1, 0, 2))
    w2rep = jnp.repeat(W2.reshape(16)[:, None], 16, axis=1)
    b2rep = jnp.repeat(b2[:, None], 16, axis=1)

    km = (path_edge_type * MAXLEN + path_positions).astype(jnp.int32)
    pl1 = path_list.astype(jnp.int32)
    pi1 = path_index.astype(jnp.int32)

    zeros = jnp.zeros((ZR, 8), f32)
    a2, d2 = _sc_sparse(y2, pw2, ewp8, w2rep, b2rep, pl1, pi1, km, zeros)
    out8 = _dense3(a2, xp, d2, ewfull)
    return out8.reshape(B, H)
